# baseline (speedup 1.0000x reference)
import functools

import jax
import jax.numpy as jnp
from jax import lax
from jax.experimental import pallas as pl
from jax.experimental.pallas import tpu as pltpu
from jax.experimental.pallas import tpu_sc as plsc

_N = 10000
_E = 320000
_DIN = 128
_HID = 256
_LAT = 64
_CTXD = 32
_W = 80
_NC = 2
_NS = 16
_NWRK = _NC * _NS
_C = 128
_NB = 4
_EPAD = 327680
_EPW = _EPAD // _NWRK
_NCH = _EPW // _C
_DUMP = _N
_NDUMP = 2048
_RPT = _N // _NS
_R = 1000
_GRID = _N // _R


def _sc_agg_body(feat_hbm, src_hbm, dst_hbm, zeros_hbm, out_hbm,
                 sidx, didx, r0, r1, r2, r3,
                 g0, g1, g2, g3, s0, s1, s2, s3, agg_sh):
  c = lax.axis_index("c")
  s = lax.axis_index("s")
  wid = c * _NS + s
  rows = (r0, r1, r2, r3)
  gsem = (g0, g1, g2, g3)
  ssem = (s0, s1, s2, s3)
  pltpu.sync_copy(zeros_hbm.at[pl.ds(s * _RPT, _RPT)],
                  agg_sh.at[pl.ds(s * _RPT, _RPT)])
  base = wid * _NCH
  pltpu.sync_copy(src_hbm.at[pl.ds(base, _NCH)], sidx)
  pltpu.sync_copy(dst_hbm.at[pl.ds(base, _NCH)], didx)
  plsc.subcore_barrier()

  del ssem

  def step(j, carry):
    pltpu.async_copy(feat_hbm.at[sidx.at[j]], rows[0], gsem[0]).wait()
    pltpu.sync_copy(rows[0], agg_sh.at[didx.at[j]], add=True)
    return carry

  lax.fori_loop(0, _NCH, step, 0)
  plsc.subcore_barrier()
  pltpu.sync_copy(agg_sh.at[pl.ds(s * _RPT, _RPT)],
                  out_hbm.at[c, pl.ds(s * _RPT, _RPT)])


@functools.cache
def _sc_agg_kernel():
  return pl.kernel(
      _sc_agg_body,
      out_type=jax.ShapeDtypeStruct((_NC, _N, _W), jnp.float32),
      mesh=plsc.VectorSubcoreMesh(core_axis_name="c", subcore_axis_name="s",
                                  num_cores=_NC, num_subcores=_NS),
      scratch_types=[
          pltpu.VMEM((_NCH, _C), jnp.int32),
          pltpu.VMEM((_NCH, _C), jnp.int32),
      ] + [pltpu.VMEM((_C, _W), jnp.float32)] * _NB
        + [pltpu.SemaphoreType.DMA] * (2 * _NB)
        + [pltpu.VMEM_SHARED((_N + _NDUMP, _W), jnp.float32)],
      compiler_params=pltpu.CompilerParams(use_tc_tiling_on_sc=False),
  )


def _sc_agg(feat, src, dst, zeros):
  return _sc_agg_kernel()(feat, src, dst, zeros)


def _gelu(x):
  return x * 0.5 * (1.0 + lax.erf(x * 0.7071067811865476))


def _enc_body(x_ref, w1, b1, w2, b2, w3p, b3p, out_ref):
  h = _gelu(jnp.dot(x_ref[...], w1[...]) + b1[0])
  h = _gelu(jnp.dot(h, w2[...]) + b2[0])
  out_ref[...] = jnp.dot(h, w3p[...]) + b3p[0]


def _full(shape):
  nd = len(shape)
  return pl.BlockSpec(shape, lambda i, _nd=nd: (0,) * _nd)


def _enc_call(x, w1, b1, w2, b2, w3p, b3p):
  return pl.pallas_call(
      _enc_body,
      grid=(_GRID,),
      in_specs=[
          pl.BlockSpec((_R, _DIN), lambda i: (i, 0)),
          _full((_DIN, _HID)), _full((1, _HID)),
          _full((_HID, _HID)), _full((1, _HID)),
          _full((_HID, _W)), _full((1, _W)),
      ],
      out_specs=pl.BlockSpec((_R, _W), lambda i: (i, 0)),
      out_shape=jax.ShapeDtypeStruct((_N, _W), jnp.float32),
  )(x, w1, b1, w2, b2, w3p, b3p)


def _layer_body(feat_ref, agg_ref, w1p, b1, w2p, b2p, out_ref):
  a = agg_ref[0] + agg_ref[1]
  lanes = lax.broadcasted_iota(jnp.int32, (_R, _W), 1)
  deg = jnp.sum(jnp.where(lanes == _LAT, a, 0.0), axis=1, keepdims=True)
  mean = a / jnp.maximum(deg, 1.0)
  h = _gelu(jnp.dot(mean, w1p[...]) + b1[0])
  out_ref[...] = feat_ref[...] + jnp.dot(h, w2p[...]) + b2p[0]


def _layer_call(feat, agg, w1p, b1, w2p, b2p):
  return pl.pallas_call(
      _layer_body,
      grid=(_GRID,),
      in_specs=[
          pl.BlockSpec((_R, _W), lambda i: (i, 0)),
          pl.BlockSpec((_NC, _R, _W), lambda i: (0, i, 0)),
          _full((_W, _HID)), _full((1, _HID)),
          _full((_HID, _W)), _full((1, _W)),
      ],
      out_specs=pl.BlockSpec((_R, _W), lambda i: (i, 0)),
      out_shape=jax.ShapeDtypeStruct((_N, _W), jnp.float32),
  )(feat, agg, w1p, b1, w2p, b2p)


def _final_body(f0, f2, f4, wfa, wfb, wfc, bf1, ln_g, ln_b, wf2, bf2,
                wd1p, bd1, wd2, bd2, wc1p, bc1, wc2, bc2,
                joint_ref, recon_ref, ctxp_ref):
  pre = (jnp.dot(f0[...], wfa[...]) + jnp.dot(f2[...], wfb[...])
         + jnp.dot(f4[...], wfc[...]) + bf1[0])
  mu = jnp.mean(pre, axis=1, keepdims=True)
  var = jnp.mean((pre - mu) ** 2, axis=1, keepdims=True)
  f = (pre - mu) / jnp.sqrt(var + 1e-5) * ln_g[0] + ln_b[0]
  f = _gelu(f)
  joint_ref[...] = jnp.dot(f, wf2[...]) + bf2[0]
  hd = _gelu(jnp.dot(f0[...], wd1p[...]) + bd1[0])
  recon_ref[...] = jnp.dot(hd, wd2[...]) + bd2[0]
  hc = _gelu(jnp.dot(f4[...], wc1p[...]) + bc1[0])
  ctxp_ref[...] = jnp.dot(hc, wc2[...]) + bc2[0]


def _final_call(f0, f2, f4, args):
  (wfa, wfb, wfc, bf1, ln_g, ln_b, wf2, bf2,
   wd1p, bd1, wd2, bd2, wc1p, bc1, wc2, bc2) = args
  return pl.pallas_call(
      _final_body,
      grid=(_GRID,),
      in_specs=[
          pl.BlockSpec((_R, _W), lambda i: (i, 0)),
          pl.BlockSpec((_R, _W), lambda i: (i, 0)),
          pl.BlockSpec((_R, _W), lambda i: (i, 0)),
          _full((_W, _HID)), _full((_W, _HID)), _full((_W, _HID)),
          _full((1, _HID)), _full((1, _HID)), _full((1, _HID)),
          _full((_HID, _LAT)), _full((1, _LAT)),
          _full((_W, _HID)), _full((1, _HID)),
          _full((_HID, _DIN)), _full((1, _DIN)),
          _full((_W, _HID)), _full((1, _HID)),
          _full((_HID, _CTXD)), _full((1, _CTXD)),
      ],
      out_specs=[
          pl.BlockSpec((_R, _LAT), lambda i: (i, 0)),
          pl.BlockSpec((_R, _DIN), lambda i: (i, 0)),
          pl.BlockSpec((_R, _CTXD), lambda i: (i, 0)),
      ],
      out_shape=[
          jax.ShapeDtypeStruct((_N, _LAT), jnp.float32),
          jax.ShapeDtypeStruct((_N, _DIN), jnp.float32),
          jax.ShapeDtypeStruct((_N, _CTXD), jnp.float32),
      ],
  )(f0, f2, f4, wfa, wfb, wfc, bf1, ln_g, ln_b, wf2, bf2,
    wd1p, bd1, wd2, bd2, wc1p, bc1, wc2, bc2)


def _pad_w(w, rows=None, cols=None, bias=None):
  r, c = w.shape
  rows = rows or r
  cols = cols or c
  out = jnp.zeros((rows, cols), jnp.float32).at[:r, :c].set(w)
  if bias is not None:
    out = out.at[_LAT, :bias.shape[0]].set(bias)
  return out


def _row(b):
  return b.reshape(1, -1)


def kernel(x, edge_index_short, edge_index_mid, params):
  p = params

  def pad_edges(ei):
    npad = _EPAD - _E
    src = jnp.concatenate([ei[0], jnp.zeros((npad,), jnp.int32)])
    dump = _DUMP + (jnp.arange(npad, dtype=jnp.int32) % _NDUMP)
    dst = jnp.concatenate([ei[1], dump])
    return src.reshape(-1, _C), dst.reshape(-1, _C)

  src_s, dst_s = pad_edges(edge_index_short)
  src_m, dst_m = pad_edges(edge_index_mid)
  zeros = jnp.zeros((_N, _W), jnp.float32)

  w3p = _pad_w(p["enc_head"]["W"], cols=_W)
  b3p = jnp.zeros((1, _W), jnp.float32).at[0, :_LAT].set(
      p["enc_head"]["b"]).at[0, _LAT].set(1.0)

  feat = _enc_call(x, p["enc_body"][0]["W"], _row(p["enc_body"][0]["b"]),
                   p["enc_body"][1]["W"], _row(p["enc_body"][1]["b"]),
                   w3p, b3p)
  feat0 = feat

  def graph_layer(feat, src, dst, gp):
    agg = _sc_agg(feat, src, dst, zeros)
    w1p = _pad_w(gp["l1"]["W"], rows=_W)
    w2p = _pad_w(gp["l2"]["W"], cols=_W)
    b2p = jnp.zeros((1, _W), jnp.float32).at[0, :_LAT].set(gp["l2"]["b"])
    return _layer_call(feat, agg, w1p, _row(gp["l1"]["b"]), w2p, b2p)

  feat = graph_layer(feat, src_s, dst_s, p["short"][0])
  feat = graph_layer(feat, src_s, dst_s, p["short"][1])
  feat2 = feat
  feat = graph_layer(feat, src_m, dst_m, p["mid"][0])
  feat = graph_layer(feat, src_m, dst_m, p["mid"][1])
  feat4 = feat

  wf = p["fusion1"]["W"]
  final_args = (
      _pad_w(wf[:_LAT], rows=_W), _pad_w(wf[_LAT:2 * _LAT], rows=_W),
      _pad_w(wf[2 * _LAT:], rows=_W), _row(p["fusion1"]["b"]),
      _row(p["ln_g"]), _row(p["ln_b"]),
      p["fusion2"]["W"], _row(p["fusion2"]["b"]),
      _pad_w(p["dec1"]["W"], rows=_W), _row(p["dec1"]["b"]),
      p["dec2"]["W"], _row(p["dec2"]["b"]),
      _pad_w(p["ctx1"]["W"], rows=_W), _row(p["ctx1"]["b"]),
      p["ctx2"]["W"], _row(p["ctx2"]["b"]),
  )
  joint, recon, ctx_pred = _final_call(feat0, feat2, feat4, final_args)

  intrinsic = feat0[:, :_LAT]
  context = feat4[:, :_LAT]
  return (intrinsic, context, joint, recon, ctx_pred)


# speedup vs baseline: 3.1710x; 3.1710x over previous
import functools

import jax
import jax.numpy as jnp
from jax import lax
from jax.experimental import pallas as pl
from jax.experimental.pallas import tpu as pltpu
from jax.experimental.pallas import tpu_sc as plsc

_N = 10000
_E = 320000
_DIN = 128
_HID = 256
_LAT = 64
_CTXD = 32
_W = 80
_NC = 2
_NS = 16
_NWRK = _NC * _NS
_C = 100
_NB = 2
_EPW = _E // _NWRK
_NCH = _EPW // _C
_RPT = _N // _NS
_R = 1000
_GRID = _N // _R


def _sc_agg_body(feat_hbm, src_hbm, dst_hbm, zeros_hbm, out_hbm,
                 sidx, didx, r0, r1, g0, g1, agg_sh):
  c = lax.axis_index("c")
  s = lax.axis_index("s")
  wid = c * _NS + s
  rows = (r0, r1)
  gsem = (g0, g1)
  pltpu.sync_copy(zeros_hbm.at[pl.ds(s * _RPT, _RPT)],
                  agg_sh.at[pl.ds(s * _RPT, _RPT)])
  base = wid * _NCH
  pltpu.sync_copy(src_hbm.at[pl.ds(base, _NCH)], sidx)
  pltpu.sync_copy(dst_hbm.at[pl.ds(base, _NCH)], didx)
  plsc.subcore_barrier()

  for b in range(2):
    pltpu.async_copy(feat_hbm.at[sidx.at[b]], rows[b], gsem[b])

  def step(t, carry):
    jb = t * 2
    for b in range(2):
      j = jb + b
      pltpu.make_async_copy(feat_hbm.at[sidx.at[j]], rows[b], gsem[b]).wait()
      pltpu.sync_copy(rows[b], agg_sh.at[didx.at[j]], add=True)
      jn = j + 2

      @pl.when(jn < _NCH)
      def _(b=b, jn=jn):
        pltpu.async_copy(feat_hbm.at[sidx.at[jn]], rows[b], gsem[b])
    return carry

  lax.fori_loop(0, _NCH // 2, step, 0)
  plsc.subcore_barrier()
  pltpu.sync_copy(agg_sh.at[pl.ds(s * _RPT, _RPT)],
                  out_hbm.at[c, pl.ds(s * _RPT, _RPT)])


@functools.cache
def _sc_agg_kernel():
  return pl.kernel(
      _sc_agg_body,
      out_type=jax.ShapeDtypeStruct((_NC, _N, _W), jnp.float32),
      mesh=plsc.VectorSubcoreMesh(core_axis_name="c", subcore_axis_name="s",
                                  num_cores=_NC, num_subcores=_NS),
      scratch_types=[
          pltpu.VMEM((_NCH, _C), jnp.int32),
          pltpu.VMEM((_NCH, _C), jnp.int32),
      ] + [pltpu.VMEM((_C, _W), jnp.float32)] * _NB
        + [pltpu.SemaphoreType.DMA] * _NB
        + [pltpu.VMEM_SHARED((_N, _W), jnp.float32)],
      compiler_params=pltpu.CompilerParams(use_tc_tiling_on_sc=False),
  )


def _sc_agg(feat, src, dst, zeros):
  return _sc_agg_kernel()(feat, src, dst, zeros)


def _gelu(x):
  return x * 0.5 * (1.0 + lax.erf(x * 0.7071067811865476))


def _enc_body(x_ref, w1, b1, w2, b2, w3p, b3p, out_ref):
  h = _gelu(jnp.dot(x_ref[...], w1[...]) + b1[0])
  h = _gelu(jnp.dot(h, w2[...]) + b2[0])
  out_ref[...] = jnp.dot(h, w3p[...]) + b3p[0]


def _full(shape):
  nd = len(shape)
  return pl.BlockSpec(shape, lambda i, _nd=nd: (0,) * _nd)


def _enc_call(x, w1, b1, w2, b2, w3p, b3p):
  return pl.pallas_call(
      _enc_body,
      grid=(_GRID,),
      in_specs=[
          pl.BlockSpec((_R, _DIN), lambda i: (i, 0)),
          _full((_DIN, _HID)), _full((1, _HID)),
          _full((_HID, _HID)), _full((1, _HID)),
          _full((_HID, _W)), _full((1, _W)),
      ],
      out_specs=pl.BlockSpec((_R, _W), lambda i: (i, 0)),
      out_shape=jax.ShapeDtypeStruct((_N, _W), jnp.float32),
  )(x, w1, b1, w2, b2, w3p, b3p)


def _layer_body(feat_ref, agg_ref, w1p, b1, w2p, b2p, out_ref):
  a = agg_ref[0] + agg_ref[1]
  lanes = lax.broadcasted_iota(jnp.int32, (_R, _W), 1)
  deg = jnp.sum(jnp.where(lanes == _LAT, a, 0.0), axis=1, keepdims=True)
  mean = a / jnp.maximum(deg, 1.0)
  h = _gelu(jnp.dot(mean, w1p[...]) + b1[0])
  out_ref[...] = feat_ref[...] + jnp.dot(h, w2p[...]) + b2p[0]


def _layer_call(feat, agg, w1p, b1, w2p, b2p):
  return pl.pallas_call(
      _layer_body,
      grid=(_GRID,),
      in_specs=[
          pl.BlockSpec((_R, _W), lambda i: (i, 0)),
          pl.BlockSpec((_NC, _R, _W), lambda i: (0, i, 0)),
          _full((_W, _HID)), _full((1, _HID)),
          _full((_HID, _W)), _full((1, _W)),
      ],
      out_specs=pl.BlockSpec((_R, _W), lambda i: (i, 0)),
      out_shape=jax.ShapeDtypeStruct((_N, _W), jnp.float32),
  )(feat, agg, w1p, b1, w2p, b2p)


def _final_body(f0, f2, f4, wfa, wfb, wfc, bf1, ln_g, ln_b, wf2, bf2,
                wd1p, bd1, wd2, bd2, wc1p, bc1, wc2, bc2,
                joint_ref, recon_ref, ctxp_ref):
  pre = (jnp.dot(f0[...], wfa[...]) + jnp.dot(f2[...], wfb[...])
         + jnp.dot(f4[...], wfc[...]) + bf1[0])
  mu = jnp.mean(pre, axis=1, keepdims=True)
  var = jnp.mean((pre - mu) ** 2, axis=1, keepdims=True)
  f = (pre - mu) / jnp.sqrt(var + 1e-5) * ln_g[0] + ln_b[0]
  f = _gelu(f)
  joint_ref[...] = jnp.dot(f, wf2[...]) + bf2[0]
  hd = _gelu(jnp.dot(f0[...], wd1p[...]) + bd1[0])
  recon_ref[...] = jnp.dot(hd, wd2[...]) + bd2[0]
  hc = _gelu(jnp.dot(f4[...], wc1p[...]) + bc1[0])
  ctxp_ref[...] = jnp.dot(hc, wc2[...]) + bc2[0]


def _final_call(f0, f2, f4, args):
  (wfa, wfb, wfc, bf1, ln_g, ln_b, wf2, bf2,
   wd1p, bd1, wd2, bd2, wc1p, bc1, wc2, bc2) = args
  return pl.pallas_call(
      _final_body,
      grid=(_GRID,),
      in_specs=[
          pl.BlockSpec((_R, _W), lambda i: (i, 0)),
          pl.BlockSpec((_R, _W), lambda i: (i, 0)),
          pl.BlockSpec((_R, _W), lambda i: (i, 0)),
          _full((_W, _HID)), _full((_W, _HID)), _full((_W, _HID)),
          _full((1, _HID)), _full((1, _HID)), _full((1, _HID)),
          _full((_HID, _LAT)), _full((1, _LAT)),
          _full((_W, _HID)), _full((1, _HID)),
          _full((_HID, _DIN)), _full((1, _DIN)),
          _full((_W, _HID)), _full((1, _HID)),
          _full((_HID, _CTXD)), _full((1, _CTXD)),
      ],
      out_specs=[
          pl.BlockSpec((_R, _LAT), lambda i: (i, 0)),
          pl.BlockSpec((_R, _DIN), lambda i: (i, 0)),
          pl.BlockSpec((_R, _CTXD), lambda i: (i, 0)),
      ],
      out_shape=[
          jax.ShapeDtypeStruct((_N, _LAT), jnp.float32),
          jax.ShapeDtypeStruct((_N, _DIN), jnp.float32),
          jax.ShapeDtypeStruct((_N, _CTXD), jnp.float32),
      ],
  )(f0, f2, f4, wfa, wfb, wfc, bf1, ln_g, ln_b, wf2, bf2,
    wd1p, bd1, wd2, bd2, wc1p, bc1, wc2, bc2)


def _pad_w(w, rows=None, cols=None, bias=None):
  r, c = w.shape
  rows = rows or r
  cols = cols or c
  out = jnp.zeros((rows, cols), jnp.float32).at[:r, :c].set(w)
  if bias is not None:
    out = out.at[_LAT, :bias.shape[0]].set(bias)
  return out


def _row(b):
  return b.reshape(1, -1)


def kernel(x, edge_index_short, edge_index_mid, params):
  p = params

  src_s = edge_index_short[0].reshape(-1, _C)
  dst_s = edge_index_short[1].reshape(-1, _C)
  src_m = edge_index_mid[0].reshape(-1, _C)
  dst_m = edge_index_mid[1].reshape(-1, _C)
  zeros = jnp.zeros((_N, _W), jnp.float32)

  w3p = _pad_w(p["enc_head"]["W"], cols=_W)
  b3p = jnp.zeros((1, _W), jnp.float32).at[0, :_LAT].set(
      p["enc_head"]["b"]).at[0, _LAT].set(1.0)

  feat = _enc_call(x, p["enc_body"][0]["W"], _row(p["enc_body"][0]["b"]),
                   p["enc_body"][1]["W"], _row(p["enc_body"][1]["b"]),
                   w3p, b3p)
  feat0 = feat

  def graph_layer(feat, src, dst, gp):
    agg = _sc_agg(feat, src, dst, zeros)
    w1p = _pad_w(gp["l1"]["W"], rows=_W)
    w2p = _pad_w(gp["l2"]["W"], cols=_W)
    b2p = jnp.zeros((1, _W), jnp.float32).at[0, :_LAT].set(gp["l2"]["b"])
    return _layer_call(feat, agg, w1p, _row(gp["l1"]["b"]), w2p, b2p)

  feat = graph_layer(feat, src_s, dst_s, p["short"][0])
  feat = graph_layer(feat, src_s, dst_s, p["short"][1])
  feat2 = feat
  feat = graph_layer(feat, src_m, dst_m, p["mid"][0])
  feat = graph_layer(feat, src_m, dst_m, p["mid"][1])
  feat4 = feat

  wf = p["fusion1"]["W"]
  final_args = (
      _pad_w(wf[:_LAT], rows=_W), _pad_w(wf[_LAT:2 * _LAT], rows=_W),
      _pad_w(wf[2 * _LAT:], rows=_W), _row(p["fusion1"]["b"]),
      _row(p["ln_g"]), _row(p["ln_b"]),
      p["fusion2"]["W"], _row(p["fusion2"]["b"]),
      _pad_w(p["dec1"]["W"], rows=_W), _row(p["dec1"]["b"]),
      p["dec2"]["W"], _row(p["dec2"]["b"]),
      _pad_w(p["ctx1"]["W"], rows=_W), _row(p["ctx1"]["b"]),
      p["ctx2"]["W"], _row(p["ctx2"]["b"]),
  )
  joint, recon, ctx_pred = _final_call(feat0, feat2, feat4, final_args)

  intrinsic = feat0[:, :_LAT]
  context = feat4[:, :_LAT]
  return (intrinsic, context, joint, recon, ctx_pred)


# speedup vs baseline: 3.3301x; 1.0502x over previous
import functools

import jax
import jax.numpy as jnp
from jax import lax
from jax.experimental import pallas as pl
from jax.experimental.pallas import tpu as pltpu
from jax.experimental.pallas import tpu_sc as plsc

_N = 10000
_E = 320000
_DIN = 128
_HID = 256
_LAT = 64
_CTXD = 32
_W = 80
_NC = 2
_NS = 16
_NWRK = _NC * _NS
_C = 125
_NB = 2
_EPW = _E // _NWRK
_NCH = _EPW // _C
_RPT = _N // _NS
_R = 1000
_GRID = _N // _R


def _sc_agg_body(feat_hbm, src_hbm, dst_hbm, zeros_hbm, out_hbm,
                 sidx, didx, r0, r1, g0, g1, agg_sh):
  c = lax.axis_index("c")
  s = lax.axis_index("s")
  wid = c * _NS + s
  rows = (r0, r1)
  gsem = (g0, g1)
  pltpu.sync_copy(zeros_hbm.at[pl.ds(s * _RPT, _RPT)],
                  agg_sh.at[pl.ds(s * _RPT, _RPT)])
  base = wid * _NCH
  pltpu.sync_copy(src_hbm.at[pl.ds(base, _NCH)], sidx)
  pltpu.sync_copy(dst_hbm.at[pl.ds(base, _NCH)], didx)
  plsc.subcore_barrier()

  for b in range(2):
    pltpu.async_copy(feat_hbm.at[sidx.at[b]], rows[b], gsem[b])

  def step(t, carry):
    jb = t * 2
    for b in range(2):
      j = jb + b
      pltpu.make_async_copy(feat_hbm.at[sidx.at[j]], rows[b], gsem[b]).wait()
      pltpu.sync_copy(rows[b], agg_sh.at[didx.at[j]], add=True)
      jn = j + 2

      @pl.when(jn < _NCH)
      def _(b=b, jn=jn):
        pltpu.async_copy(feat_hbm.at[sidx.at[jn]], rows[b], gsem[b])
    return carry

  lax.fori_loop(0, _NCH // 2, step, 0)
  plsc.subcore_barrier()
  pltpu.sync_copy(agg_sh.at[pl.ds(s * _RPT, _RPT)],
                  out_hbm.at[c, pl.ds(s * _RPT, _RPT)])


@functools.cache
def _sc_agg_kernel():
  return pl.kernel(
      _sc_agg_body,
      out_type=jax.ShapeDtypeStruct((_NC, _N, _W), jnp.float32),
      mesh=plsc.VectorSubcoreMesh(core_axis_name="c", subcore_axis_name="s",
                                  num_cores=_NC, num_subcores=_NS),
      scratch_types=[
          pltpu.VMEM((_NCH, _C), jnp.int32),
          pltpu.VMEM((_NCH, _C), jnp.int32),
      ] + [pltpu.VMEM((_C, _W), jnp.float32)] * _NB
        + [pltpu.SemaphoreType.DMA] * _NB
        + [pltpu.VMEM_SHARED((_N, _W), jnp.float32)],
      compiler_params=pltpu.CompilerParams(use_tc_tiling_on_sc=False),
  )


def _sc_agg(feat, src, dst, zeros):
  return _sc_agg_kernel()(feat, src, dst, zeros)


def _gelu(x):
  return x * 0.5 * (1.0 + lax.erf(x * 0.7071067811865476))


def _enc_body(x_ref, w1, b1, w2, b2, w3p, b3p, out_ref):
  h = _gelu(jnp.dot(x_ref[...], w1[...]) + b1[0])
  h = _gelu(jnp.dot(h, w2[...]) + b2[0])
  out_ref[...] = jnp.dot(h, w3p[...]) + b3p[0]


def _full(shape):
  nd = len(shape)
  return pl.BlockSpec(shape, lambda i, _nd=nd: (0,) * _nd)


def _enc_call(x, w1, b1, w2, b2, w3p, b3p):
  return pl.pallas_call(
      _enc_body,
      grid=(_GRID,),
      in_specs=[
          pl.BlockSpec((_R, _DIN), lambda i: (i, 0)),
          _full((_DIN, _HID)), _full((1, _HID)),
          _full((_HID, _HID)), _full((1, _HID)),
          _full((_HID, _W)), _full((1, _W)),
      ],
      out_specs=pl.BlockSpec((_R, _W), lambda i: (i, 0)),
      out_shape=jax.ShapeDtypeStruct((_N, _W), jnp.float32),
  )(x, w1, b1, w2, b2, w3p, b3p)


def _layer_body(feat_ref, agg_ref, w1p, b1, w2p, b2p, out_ref):
  a = agg_ref[0] + agg_ref[1]
  lanes = lax.broadcasted_iota(jnp.int32, (_R, _W), 1)
  deg = jnp.sum(jnp.where(lanes == _LAT, a, 0.0), axis=1, keepdims=True)
  mean = a / jnp.maximum(deg, 1.0)
  h = _gelu(jnp.dot(mean, w1p[...]) + b1[0])
  out_ref[...] = feat_ref[...] + jnp.dot(h, w2p[...]) + b2p[0]


def _layer_call(feat, agg, w1p, b1, w2p, b2p):
  return pl.pallas_call(
      _layer_body,
      grid=(_GRID,),
      in_specs=[
          pl.BlockSpec((_R, _W), lambda i: (i, 0)),
          pl.BlockSpec((_NC, _R, _W), lambda i: (0, i, 0)),
          _full((_W, _HID)), _full((1, _HID)),
          _full((_HID, _W)), _full((1, _W)),
      ],
      out_specs=pl.BlockSpec((_R, _W), lambda i: (i, 0)),
      out_shape=jax.ShapeDtypeStruct((_N, _W), jnp.float32),
  )(feat, agg, w1p, b1, w2p, b2p)


def _final_body(f0, f2, f4, wfa, wfb, wfc, bf1, ln_g, ln_b, wf2, bf2,
                wd1p, bd1, wd2, bd2, wc1p, bc1, wc2, bc2,
                joint_ref, recon_ref, ctxp_ref):
  pre = (jnp.dot(f0[...], wfa[...]) + jnp.dot(f2[...], wfb[...])
         + jnp.dot(f4[...], wfc[...]) + bf1[0])
  mu = jnp.mean(pre, axis=1, keepdims=True)
  var = jnp.mean((pre - mu) ** 2, axis=1, keepdims=True)
  f = (pre - mu) / jnp.sqrt(var + 1e-5) * ln_g[0] + ln_b[0]
  f = _gelu(f)
  joint_ref[...] = jnp.dot(f, wf2[...]) + bf2[0]
  hd = _gelu(jnp.dot(f0[...], wd1p[...]) + bd1[0])
  recon_ref[...] = jnp.dot(hd, wd2[...]) + bd2[0]
  hc = _gelu(jnp.dot(f4[...], wc1p[...]) + bc1[0])
  ctxp_ref[...] = jnp.dot(hc, wc2[...]) + bc2[0]


def _final_call(f0, f2, f4, args):
  (wfa, wfb, wfc, bf1, ln_g, ln_b, wf2, bf2,
   wd1p, bd1, wd2, bd2, wc1p, bc1, wc2, bc2) = args
  return pl.pallas_call(
      _final_body,
      grid=(_GRID,),
      in_specs=[
          pl.BlockSpec((_R, _W), lambda i: (i, 0)),
          pl.BlockSpec((_R, _W), lambda i: (i, 0)),
          pl.BlockSpec((_R, _W), lambda i: (i, 0)),
          _full((_W, _HID)), _full((_W, _HID)), _full((_W, _HID)),
          _full((1, _HID)), _full((1, _HID)), _full((1, _HID)),
          _full((_HID, _LAT)), _full((1, _LAT)),
          _full((_W, _HID)), _full((1, _HID)),
          _full((_HID, _DIN)), _full((1, _DIN)),
          _full((_W, _HID)), _full((1, _HID)),
          _full((_HID, _CTXD)), _full((1, _CTXD)),
      ],
      out_specs=[
          pl.BlockSpec((_R, _LAT), lambda i: (i, 0)),
          pl.BlockSpec((_R, _DIN), lambda i: (i, 0)),
          pl.BlockSpec((_R, _CTXD), lambda i: (i, 0)),
      ],
      out_shape=[
          jax.ShapeDtypeStruct((_N, _LAT), jnp.float32),
          jax.ShapeDtypeStruct((_N, _DIN), jnp.float32),
          jax.ShapeDtypeStruct((_N, _CTXD), jnp.float32),
      ],
  )(f0, f2, f4, wfa, wfb, wfc, bf1, ln_g, ln_b, wf2, bf2,
    wd1p, bd1, wd2, bd2, wc1p, bc1, wc2, bc2)


def _pad_w(w, rows=None, cols=None, bias=None):
  r, c = w.shape
  rows = rows or r
  cols = cols or c
  out = jnp.zeros((rows, cols), jnp.float32).at[:r, :c].set(w)
  if bias is not None:
    out = out.at[_LAT, :bias.shape[0]].set(bias)
  return out


def _row(b):
  return b.reshape(1, -1)


def kernel(x, edge_index_short, edge_index_mid, params):
  p = params

  src_s = edge_index_short[0].reshape(-1, _C)
  dst_s = edge_index_short[1].reshape(-1, _C)
  src_m = edge_index_mid[0].reshape(-1, _C)
  dst_m = edge_index_mid[1].reshape(-1, _C)
  zeros = jnp.zeros((_N, _W), jnp.float32)

  w3p = _pad_w(p["enc_head"]["W"], cols=_W)
  b3p = jnp.zeros((1, _W), jnp.float32).at[0, :_LAT].set(
      p["enc_head"]["b"]).at[0, _LAT].set(1.0)

  feat = _enc_call(x, p["enc_body"][0]["W"], _row(p["enc_body"][0]["b"]),
                   p["enc_body"][1]["W"], _row(p["enc_body"][1]["b"]),
                   w3p, b3p)
  feat0 = feat

  def graph_layer(feat, src, dst, gp):
    agg = _sc_agg(feat, src, dst, zeros)
    w1p = _pad_w(gp["l1"]["W"], rows=_W)
    w2p = _pad_w(gp["l2"]["W"], cols=_W)
    b2p = jnp.zeros((1, _W), jnp.float32).at[0, :_LAT].set(gp["l2"]["b"])
    return _layer_call(feat, agg, w1p, _row(gp["l1"]["b"]), w2p, b2p)

  feat = graph_layer(feat, src_s, dst_s, p["short"][0])
  feat = graph_layer(feat, src_s, dst_s, p["short"][1])
  feat2 = feat
  feat = graph_layer(feat, src_m, dst_m, p["mid"][0])
  feat = graph_layer(feat, src_m, dst_m, p["mid"][1])
  feat4 = feat

  wf = p["fusion1"]["W"]
  final_args = (
      _pad_w(wf[:_LAT], rows=_W), _pad_w(wf[_LAT:2 * _LAT], rows=_W),
      _pad_w(wf[2 * _LAT:], rows=_W), _row(p["fusion1"]["b"]),
      _row(p["ln_g"]), _row(p["ln_b"]),
      p["fusion2"]["W"], _row(p["fusion2"]["b"]),
      _pad_w(p["dec1"]["W"], rows=_W), _row(p["dec1"]["b"]),
      p["dec2"]["W"], _row(p["dec2"]["b"]),
      _pad_w(p["ctx1"]["W"], rows=_W), _row(p["ctx1"]["b"]),
      p["ctx2"]["W"], _row(p["ctx2"]["b"]),
  )
  joint, recon, ctx_pred = _final_call(feat0, feat2, feat4, final_args)

  intrinsic = feat0[:, :_LAT]
  context = feat4[:, :_LAT]
  return (intrinsic, context, joint, recon, ctx_pred)


# speedup vs baseline: 3.4629x; 1.0399x over previous
import functools

import jax
import jax.numpy as jnp
from jax import lax
from jax.experimental import pallas as pl
from jax.experimental.pallas import tpu as pltpu
from jax.experimental.pallas import tpu_sc as plsc

_N = 10000
_E = 320000
_DIN = 128
_HID = 256
_LAT = 64
_CTXD = 32
_W = 80
_NC = 2
_NS = 16
_NWRK = _NC * _NS
_C = 125
_NB = 4
_EPW = _E // _NWRK
_NCH = _EPW // _C
_RPT = _N // _NS
_R = 1000
_GRID = _N // _R


def _sc_agg_body(feat_hbm, src_hbm, dst_hbm, zeros_hbm, out_hbm,
                 sidx, didx, r0, r1, r2, r3, g0, g1, g2, g3,
                 s0, s1, s2, s3, agg_sh):
  c = lax.axis_index("c")
  s = lax.axis_index("s")
  wid = c * _NS + s
  rows = (r0, r1, r2, r3)
  gsem = (g0, g1, g2, g3)
  ssem = (s0, s1, s2, s3)
  pltpu.sync_copy(zeros_hbm.at[pl.ds(s * _RPT, _RPT)],
                  agg_sh.at[pl.ds(s * _RPT, _RPT)])
  base = wid * _NCH
  pltpu.sync_copy(src_hbm.at[pl.ds(base, _NCH)], sidx)
  pltpu.sync_copy(dst_hbm.at[pl.ds(base, _NCH)], didx)
  plsc.subcore_barrier()

  for b in range(_NB):
    pltpu.async_copy(feat_hbm.at[sidx.at[b]], rows[b], gsem[b])

  def step(t, carry):
    jb = t * _NB
    for b in range(_NB):
      j = jb + b
      pltpu.make_async_copy(feat_hbm.at[sidx.at[j]], rows[b], gsem[b]).wait()
      pltpu.async_copy(rows[b], agg_sh.at[didx.at[j]], ssem[b], add=True)
    for b in range(_NB):
      jn = jb + _NB + b

      @pl.when(jn < _NCH)
      def _(b=b, jn=jn):
        pltpu.make_async_copy(rows[b], agg_sh.at[didx.at[jn]], ssem[b]).wait()
        pltpu.async_copy(feat_hbm.at[sidx.at[jn]], rows[b], gsem[b])
    return carry

  lax.fori_loop(0, _NCH // _NB, step, 0)
  for b in range(_NB):
    pltpu.make_async_copy(rows[b], agg_sh.at[didx.at[0]], ssem[b]).wait()
  plsc.subcore_barrier()
  pltpu.sync_copy(agg_sh.at[pl.ds(s * _RPT, _RPT)],
                  out_hbm.at[c, pl.ds(s * _RPT, _RPT)])


@functools.cache
def _sc_agg_kernel():
  return pl.kernel(
      _sc_agg_body,
      out_type=jax.ShapeDtypeStruct((_NC, _N, _W), jnp.float32),
      mesh=plsc.VectorSubcoreMesh(core_axis_name="c", subcore_axis_name="s",
                                  num_cores=_NC, num_subcores=_NS),
      scratch_types=[
          pltpu.VMEM((_NCH, _C), jnp.int32),
          pltpu.VMEM((_NCH, _C), jnp.int32),
      ] + [pltpu.VMEM((_C, _W), jnp.float32)] * _NB
        + [pltpu.SemaphoreType.DMA] * (2 * _NB)
        + [pltpu.VMEM_SHARED((_N, _W), jnp.float32)],
      compiler_params=pltpu.CompilerParams(use_tc_tiling_on_sc=False),
  )


def _sc_agg(feat, src, dst, zeros):
  return _sc_agg_kernel()(feat, src, dst, zeros)


def _gelu(x):
  return x * 0.5 * (1.0 + lax.erf(x * 0.7071067811865476))


def _enc_body(x_ref, w1, b1, w2, b2, w3p, b3p, out_ref):
  h = _gelu(jnp.dot(x_ref[...], w1[...]) + b1[0])
  h = _gelu(jnp.dot(h, w2[...]) + b2[0])
  out_ref[...] = jnp.dot(h, w3p[...]) + b3p[0]


def _full(shape):
  nd = len(shape)
  return pl.BlockSpec(shape, lambda i, _nd=nd: (0,) * _nd)


def _enc_call(x, w1, b1, w2, b2, w3p, b3p):
  return pl.pallas_call(
      _enc_body,
      grid=(_GRID,),
      in_specs=[
          pl.BlockSpec((_R, _DIN), lambda i: (i, 0)),
          _full((_DIN, _HID)), _full((1, _HID)),
          _full((_HID, _HID)), _full((1, _HID)),
          _full((_HID, _W)), _full((1, _W)),
      ],
      out_specs=pl.BlockSpec((_R, _W), lambda i: (i, 0)),
      out_shape=jax.ShapeDtypeStruct((_N, _W), jnp.float32),
  )(x, w1, b1, w2, b2, w3p, b3p)


def _layer_body(feat_ref, agg_ref, w1p, b1, w2p, b2p, out_ref):
  a = agg_ref[0] + agg_ref[1]
  lanes = lax.broadcasted_iota(jnp.int32, (_R, _W), 1)
  deg = jnp.sum(jnp.where(lanes == _LAT, a, 0.0), axis=1, keepdims=True)
  mean = a / jnp.maximum(deg, 1.0)
  h = _gelu(jnp.dot(mean, w1p[...]) + b1[0])
  out_ref[...] = feat_ref[...] + jnp.dot(h, w2p[...]) + b2p[0]


def _layer_call(feat, agg, w1p, b1, w2p, b2p):
  return pl.pallas_call(
      _layer_body,
      grid=(_GRID,),
      in_specs=[
          pl.BlockSpec((_R, _W), lambda i: (i, 0)),
          pl.BlockSpec((_NC, _R, _W), lambda i: (0, i, 0)),
          _full((_W, _HID)), _full((1, _HID)),
          _full((_HID, _W)), _full((1, _W)),
      ],
      out_specs=pl.BlockSpec((_R, _W), lambda i: (i, 0)),
      out_shape=jax.ShapeDtypeStruct((_N, _W), jnp.float32),
  )(feat, agg, w1p, b1, w2p, b2p)


def _final_body(f0, f2, f4, wfa, wfb, wfc, bf1, ln_g, ln_b, wf2, bf2,
                wd1p, bd1, wd2, bd2, wc1p, bc1, wc2, bc2,
                joint_ref, recon_ref, ctxp_ref):
  pre = (jnp.dot(f0[...], wfa[...]) + jnp.dot(f2[...], wfb[...])
         + jnp.dot(f4[...], wfc[...]) + bf1[0])
  mu = jnp.mean(pre, axis=1, keepdims=True)
  var = jnp.mean((pre - mu) ** 2, axis=1, keepdims=True)
  f = (pre - mu) / jnp.sqrt(var + 1e-5) * ln_g[0] + ln_b[0]
  f = _gelu(f)
  joint_ref[...] = jnp.dot(f, wf2[...]) + bf2[0]
  hd = _gelu(jnp.dot(f0[...], wd1p[...]) + bd1[0])
  recon_ref[...] = jnp.dot(hd, wd2[...]) + bd2[0]
  hc = _gelu(jnp.dot(f4[...], wc1p[...]) + bc1[0])
  ctxp_ref[...] = jnp.dot(hc, wc2[...]) + bc2[0]


def _final_call(f0, f2, f4, args):
  (wfa, wfb, wfc, bf1, ln_g, ln_b, wf2, bf2,
   wd1p, bd1, wd2, bd2, wc1p, bc1, wc2, bc2) = args
  return pl.pallas_call(
      _final_body,
      grid=(_GRID,),
      in_specs=[
          pl.BlockSpec((_R, _W), lambda i: (i, 0)),
          pl.BlockSpec((_R, _W), lambda i: (i, 0)),
          pl.BlockSpec((_R, _W), lambda i: (i, 0)),
          _full((_W, _HID)), _full((_W, _HID)), _full((_W, _HID)),
          _full((1, _HID)), _full((1, _HID)), _full((1, _HID)),
          _full((_HID, _LAT)), _full((1, _LAT)),
          _full((_W, _HID)), _full((1, _HID)),
          _full((_HID, _DIN)), _full((1, _DIN)),
          _full((_W, _HID)), _full((1, _HID)),
          _full((_HID, _CTXD)), _full((1, _CTXD)),
      ],
      out_specs=[
          pl.BlockSpec((_R, _LAT), lambda i: (i, 0)),
          pl.BlockSpec((_R, _DIN), lambda i: (i, 0)),
          pl.BlockSpec((_R, _CTXD), lambda i: (i, 0)),
      ],
      out_shape=[
          jax.ShapeDtypeStruct((_N, _LAT), jnp.float32),
          jax.ShapeDtypeStruct((_N, _DIN), jnp.float32),
          jax.ShapeDtypeStruct((_N, _CTXD), jnp.float32),
      ],
  )(f0, f2, f4, wfa, wfb, wfc, bf1, ln_g, ln_b, wf2, bf2,
    wd1p, bd1, wd2, bd2, wc1p, bc1, wc2, bc2)


def _pad_w(w, rows=None, cols=None, bias=None):
  r, c = w.shape
  rows = rows or r
  cols = cols or c
  out = jnp.zeros((rows, cols), jnp.float32).at[:r, :c].set(w)
  if bias is not None:
    out = out.at[_LAT, :bias.shape[0]].set(bias)
  return out


def _row(b):
  return b.reshape(1, -1)


def kernel(x, edge_index_short, edge_index_mid, params):
  p = params

  src_s = edge_index_short[0].reshape(-1, _C)
  dst_s = edge_index_short[1].reshape(-1, _C)
  src_m = edge_index_mid[0].reshape(-1, _C)
  dst_m = edge_index_mid[1].reshape(-1, _C)
  zeros = jnp.zeros((_N, _W), jnp.float32)

  w3p = _pad_w(p["enc_head"]["W"], cols=_W)
  b3p = jnp.zeros((1, _W), jnp.float32).at[0, :_LAT].set(
      p["enc_head"]["b"]).at[0, _LAT].set(1.0)

  feat = _enc_call(x, p["enc_body"][0]["W"], _row(p["enc_body"][0]["b"]),
                   p["enc_body"][1]["W"], _row(p["enc_body"][1]["b"]),
                   w3p, b3p)
  feat0 = feat

  def graph_layer(feat, src, dst, gp):
    agg = _sc_agg(feat, src, dst, zeros)
    w1p = _pad_w(gp["l1"]["W"], rows=_W)
    w2p = _pad_w(gp["l2"]["W"], cols=_W)
    b2p = jnp.zeros((1, _W), jnp.float32).at[0, :_LAT].set(gp["l2"]["b"])
    return _layer_call(feat, agg, w1p, _row(gp["l1"]["b"]), w2p, b2p)

  feat = graph_layer(feat, src_s, dst_s, p["short"][0])
  feat = graph_layer(feat, src_s, dst_s, p["short"][1])
  feat2 = feat
  feat = graph_layer(feat, src_m, dst_m, p["mid"][0])
  feat = graph_layer(feat, src_m, dst_m, p["mid"][1])
  feat4 = feat

  wf = p["fusion1"]["W"]
  final_args = (
      _pad_w(wf[:_LAT], rows=_W), _pad_w(wf[_LAT:2 * _LAT], rows=_W),
      _pad_w(wf[2 * _LAT:], rows=_W), _row(p["fusion1"]["b"]),
      _row(p["ln_g"]), _row(p["ln_b"]),
      p["fusion2"]["W"], _row(p["fusion2"]["b"]),
      _pad_w(p["dec1"]["W"], rows=_W), _row(p["dec1"]["b"]),
      p["dec2"]["W"], _row(p["dec2"]["b"]),
      _pad_w(p["ctx1"]["W"], rows=_W), _row(p["ctx1"]["b"]),
      p["ctx2"]["W"], _row(p["ctx2"]["b"]),
  )
  joint, recon, ctx_pred = _final_call(feat0, feat2, feat4, final_args)

  intrinsic = feat0[:, :_LAT]
  context = feat4[:, :_LAT]
  return (intrinsic, context, joint, recon, ctx_pred)
